# baseline (speedup 1.0000x reference)
import functools

import jax
import jax.numpy as jnp
from jax import lax
from jax.experimental import pallas as pl
from jax.experimental.pallas import tpu as pltpu
from jax.experimental.pallas import tpu_sc as plsc

N = 4096
H = 256
NHEAD = 8
HDIM = 32
RBF = 64
SLOTS = 32
E = N * SLOTS
CUT = 5.0
_INTERP = False


def _sc_gather(table, idx):
    R, D = table.shape
    B = idx.shape[0]
    NC, NW = 2, 32
    RPW = B // NW
    CK = min(128, RPW)
    while 2 * CK * D * 4 > 470_000:
        CK //= 2
    nch = RPW // CK
    mesh = plsc.VectorSubcoreMesh(core_axis_name="c", subcore_axis_name="s")

    @functools.partial(
        pl.kernel, mesh=mesh,
        out_type=jax.ShapeDtypeStruct((B, D), jnp.float32),
        scratch_types=[
            pltpu.VMEM((RPW,), jnp.int32),
            pltpu.VMEM((2, CK, D), jnp.float32),
            pltpu.SemaphoreType.DMA((2,)),
        ],
    )
    def k(table_hbm, idx_hbm, out_hbm, idx_v, rows_v, sem):
        wid = lax.axis_index("s") * NC + lax.axis_index("c")
        base0 = wid * RPW
        pltpu.sync_copy(idx_hbm.at[pl.ds(base0, RPW)], idx_v)

        def start(c, slot):
            pltpu.async_copy(table_hbm.at[idx_v.at[pl.ds(c * CK, CK)]],
                             rows_v.at[slot], sem.at[slot])

        start(0, 0)

        def body(c, carry):
            slot = lax.rem(c, 2)

            @pl.when(c + 1 < nch)
            def _():
                start(c + 1, lax.rem(c + 1, 2))

            pltpu.make_async_copy(
                table_hbm.at[idx_v.at[pl.ds(c * CK, CK)]],
                rows_v.at[slot], sem.at[slot]).wait()
            pltpu.sync_copy(rows_v.at[slot],
                            out_hbm.at[pl.ds(base0 + c * CK, CK)])
            return carry

        lax.fori_loop(0, nch, body, 0)

    return k(table, idx)


def _silu(x):
    return x / (1.0 + jnp.exp(-x))


def _nbr_kernel(pos8, posT, batch_c, batch_cT):
    RB = 256
    W = 1024
    INF = float(jnp.inf)

    def body(pos_ref, posT_ref, bc_ref, bcT_ref, nbr_ref, msk_ref):
        b = pl.program_id(0)
        r0 = b * RB
        bcol = bc_ref[...]
        rsel = lax.broadcasted_iota(jnp.int32, (N, 1), 0) == r0
        b0 = jnp.sum(jnp.where(rsel, bcol, 0))
        w = jnp.sum((bcol < b0).astype(jnp.int32))
        w = jnp.minimum((w // 128) * 128, N - W)
        w = pl.multiple_of(w, 128)

        pr = pos_ref[pl.ds(r0, RB), :]
        pcT = posT_ref[:, pl.ds(w, W)]
        sq_r = pr[:, 0:1] ** 2 + pr[:, 1:2] ** 2 + pr[:, 2:3] ** 2
        sq_c = pcT[0:1, :] ** 2 + pcT[1:2, :] ** 2 + pcT[2:3, :] ** 2
        dot = jax.lax.dot_general(
            pr, pcT, (((1,), (0,)), ((), ())),
            precision=jax.lax.Precision.HIGHEST)
        d2 = (sq_r + sq_c) - 2.0 * dot
        d = jnp.sqrt(jnp.maximum(d2, 0.0))

        br = bc_ref[pl.ds(r0, RB), :]
        bcT_w = bcT_ref[:, pl.ds(w, W)]
        colidx = w + lax.broadcasted_iota(jnp.int32, (1, W), 1)
        rowidx = r0 + lax.broadcasted_iota(jnp.int32, (RB, 1), 0)
        valid = (br == bcT_w) & (d < CUT) & (colidx != rowidx)
        dw = jnp.where(valid, d, INF)

        lane = lax.broadcasted_iota(jnp.int32, (1, SLOTS), 1)
        BIG = jnp.int32(2 ** 30)

        def step(t, carry):
            dwc, nbr, msk = carry
            m = jnp.min(dwc, axis=1, keepdims=True)
            am = jnp.min(jnp.where(dwc == m, colidx, BIG),
                         axis=1, keepdims=True)
            has = m < INF
            amc = jnp.where(has, am, 0)
            onec = lane == t
            nbr = jnp.where(onec, amc, nbr)
            msk = jnp.where(onec, has.astype(jnp.float32), msk)
            dwc = jnp.where(colidx == am, INF, dwc)
            return dwc, nbr, msk

        nbr0 = jnp.zeros((RB, SLOTS), jnp.int32)
        msk0 = jnp.zeros((RB, SLOTS), jnp.float32)
        _, nbr, msk = lax.fori_loop(0, SLOTS, step, (dw, nbr0, msk0))
        nbr_ref[...] = nbr
        msk_ref[...] = msk

    full = lambda shape: pl.BlockSpec(shape, lambda b: tuple(0 for _ in shape))
    return pl.pallas_call(
        body,
        grid=(N // RB,),
        in_specs=[full((N, 8)), full((8, N)), full((N, 1)), full((1, N))],
        out_specs=[pl.BlockSpec((RB, SLOTS), lambda b: (b, 0)),
                   pl.BlockSpec((RB, SLOTS), lambda b: (b, 0))],
        out_shape=[jax.ShapeDtypeStruct((N, SLOTS), jnp.int32),
                   jax.ShapeDtypeStruct((N, SLOTS), jnp.float32)],
        interpret=_INTERP,
    )(pos8, posT, batch_c, batch_cT)


def _edge_pre_kernel(pos8, pos_s3, means, betas):
    NB = 128
    EB = NB * SLOTS

    def body(pos_ref, ps_ref, mean_ref, beta_ref, f_ref, d4_ref, c_ref):
        p = pos_ref[...]
        ps = ps_ref[...]
        prep = jnp.broadcast_to(p[:, None, :], (NB, SLOTS, 8))
        prep = prep.reshape(EB, 8)
        psf = ps.reshape(EB, 128)
        evx = psf[:, 0:1] - prep[:, 0:1]
        evy = psf[:, 1:2] - prep[:, 1:2]
        evz = psf[:, 2:3] - prep[:, 2:3]
        wgt = jnp.sqrt(evx * evx + evy * evy + evz * evz)
        wsafe = jnp.where(wgt > 0.0, wgt, 1.0)
        zero = jnp.zeros((EB, 1), jnp.float32)
        d4_ref[...] = jnp.concatenate(
            [evx / wsafe, evy / wsafe, evz / wsafe, zero], axis=1)
        cosc = 0.5 * (jnp.cos(wgt * (jnp.pi / CUT)) + 1.0)
        cosc = cosc * (wgt < CUT).astype(jnp.float32)
        c_ref[...] = cosc
        mu = mean_ref[...]
        be = beta_ref[...]
        t = jnp.exp(-wgt) - mu
        f_ref[...] = cosc * jnp.exp(-be * t * t)

    full = lambda shape: pl.BlockSpec(shape, lambda b: tuple(0 for _ in shape))
    return pl.pallas_call(
        body,
        grid=(N // NB,),
        in_specs=[pl.BlockSpec((NB, 8), lambda b: (b, 0)),
                  pl.BlockSpec((NB, SLOTS, 128), lambda b: (b, 0, 0)),
                  full((1, RBF)), full((1, RBF))],
        out_specs=[pl.BlockSpec((EB, RBF), lambda b: (b, 0)),
                   pl.BlockSpec((EB, 4), lambda b: (b, 0)),
                   pl.BlockSpec((EB, 1), lambda b: (b, 0))],
        out_shape=[jax.ShapeDtypeStruct((E, RBF), jnp.float32),
                   jax.ShapeDtypeStruct((E, 4), jnp.float32),
                   jax.ShapeDtypeStruct((E, 1), jnp.float32)],
        interpret=_INTERP,
    )(pos8, pos_s3, means, betas)


def _nbr_embed_kernel(f, xz_s, cvec, mvec, x0, dw, db, cw, cb):
    NB = 128
    EB = NB * SLOTS

    def body(f_ref, xz_ref, c_ref, m_ref, x0_ref, dw_ref, db_ref,
             cw_ref, cb_ref, out_ref):
        Wm = jnp.dot(f_ref[...], dw_ref[...]) + db_ref[...]
        Wm = Wm * c_ref[...] * m_ref[...]
        msgs = Wm * xz_ref[...]
        agg = msgs.reshape(NB, SLOTS, H).sum(axis=1)
        xcat = jnp.concatenate([x0_ref[...], agg], axis=1)
        out_ref[...] = jnp.dot(xcat, cw_ref[...]) + cb_ref[...]

    full = lambda shape: pl.BlockSpec(shape, lambda b: tuple(0 for _ in shape))
    return pl.pallas_call(
        body,
        grid=(N // NB,),
        in_specs=[pl.BlockSpec((EB, RBF), lambda b: (b, 0)),
                  pl.BlockSpec((EB, H), lambda b: (b, 0)),
                  pl.BlockSpec((EB, 1), lambda b: (b, 0)),
                  pl.BlockSpec((EB, 1), lambda b: (b, 0)),
                  pl.BlockSpec((NB, H), lambda b: (b, 0)),
                  full((RBF, H)), full((1, H)),
                  full((2 * H, H)), full((1, H))],
        out_specs=pl.BlockSpec((NB, H), lambda b: (b, 0)),
        out_shape=jax.ShapeDtypeStruct((N, H), jnp.float32),
        interpret=_INTERP,
    )(f, xz_s, cvec, mvec, x0, dw, db, cw, cb)


def _layer_norm(x, w, b):
    mu = jnp.mean(x, axis=-1, keepdims=True)
    var = jnp.mean((x - mu) ** 2, axis=-1, keepdims=True)
    return (x - mu) / jnp.sqrt(var + 1e-5) * w + b


def _layer_pre_kernel(x, vecS, lnw, lnb, qw, qb, kw, kb, vw, vb, vecw):
    NB = 256

    def body(x_ref, vec_ref, lnw_ref, lnb_ref, qw_ref, qb_ref, kw_ref,
             kb_ref, vw_ref, vb_ref, vecw_ref,
             q_ref, tab_ref, vdot_ref, vec3_ref):
        xn = _layer_norm(x_ref[...], lnw_ref[...], lnb_ref[...])
        q = jnp.dot(xn, qw_ref[...]) + qb_ref[...]
        k = jnp.dot(xn, kw_ref[...]) + kb_ref[...]
        v = jnp.dot(xn, vw_ref[...]) + vb_ref[...]
        q_ref[...] = q
        vec = vec_ref[...]
        vdot = jnp.zeros((NB, H), jnp.float32)
        vecw_v = vecw_ref[...]
        for a in range(3):
            vp = jnp.dot(vec[a], vecw_v)
            vdot = vdot + vp[:, :H] * vp[:, H:2 * H]
            vec3_ref[a, :, :] = vp[:, 2 * H:]
        vdot_ref[...] = vdot
        tab_ref[...] = jnp.concatenate(
            [k, v, vec[0], vec[1], vec[2]], axis=1)

    full = lambda shape: pl.BlockSpec(shape, lambda b: tuple(0 for _ in shape))
    return pl.pallas_call(
        body,
        grid=(N // NB,),
        in_specs=[pl.BlockSpec((NB, H), lambda b: (b, 0)),
                  pl.BlockSpec((3, NB, H), lambda b: (0, b, 0)),
                  full((1, H)), full((1, H)),
                  full((H, H)), full((1, H)),
                  full((H, H)), full((1, H)),
                  full((H, 3 * H)), full((1, 3 * H)),
                  full((H, 3 * H))],
        out_specs=[pl.BlockSpec((NB, H), lambda b: (b, 0)),
                   pl.BlockSpec((NB, 7 * H), lambda b: (b, 0)),
                   pl.BlockSpec((NB, H), lambda b: (b, 0)),
                   pl.BlockSpec((3, NB, H), lambda b: (0, b, 0))],
        out_shape=[jax.ShapeDtypeStruct((N, H), jnp.float32),
                   jax.ShapeDtypeStruct((N, 7 * H), jnp.float32),
                   jax.ShapeDtypeStruct((N, H), jnp.float32),
                   jax.ShapeDtypeStruct((3, N, H), jnp.float32)],
        interpret=_INTERP,
    )(x, vecS, lnw, lnb, qw, qb, kw, kb, vw, vb, vecw)


def _layer_edge_kernel(q, g, f, cvec, mvec, d4, vdot, vec3S, x, vecS,
                       dkw, dkb, dvw, dvb, ow, ob):
    NB = 32
    EB = NB * SLOTS

    def body(q_ref, g_ref, f_ref, c_ref, m_ref, d4_ref, vdot_ref,
             vec3_ref, x_ref, vec_ref, dkw_ref, dkb_ref, dvw_ref,
             dvb_ref, ow_ref, ob_ref, xo_ref, veco_ref):
        fv = f_ref[...]
        dk = _silu(jnp.dot(fv, dkw_ref[...]) + dkb_ref[...])
        dv = _silu(jnp.dot(fv, dvw_ref[...]) + dvb_ref[...])
        g = g_ref[...]
        qv = q_ref[...]
        qrep = jnp.broadcast_to(qv[:, None, :], (NB, SLOTS, H)).reshape(EB, H)
        prod = g[:, :H] * dk * qrep
        hsel = (lax.broadcasted_iota(jnp.int32, (H, NHEAD), 0) // HDIM ==
                lax.broadcasted_iota(jnp.int32, (H, NHEAD), 1)
                ).astype(jnp.float32)
        attn = jnp.dot(prod, hsel)
        attn = _silu(attn) * c_ref[...] * m_ref[...]
        attn_exp = jnp.dot(attn, hsel.T)
        vj = g[:, H:4 * H] * dv
        xmsg = vj[:, :H] * attn_exp
        x_agg = xmsg.reshape(NB, SLOTS, H).sum(axis=1)
        o = jnp.dot(x_agg, ow_ref[...]) + ob_ref[...]
        o1, o2, o3 = o[:, :H], o[:, H:2 * H], o[:, 2 * H:]
        xo_ref[...] = x_ref[...] + vdot_ref[...] * o2 + o3
        v1m = vj[:, H:2 * H]
        v2m = vj[:, 2 * H:]
        mv = m_ref[...]
        d4v = d4_ref[...]
        for a in range(3):
            vmsg = (g[:, (4 + a) * H:(5 + a) * H] * v1m
                    + v2m * d4v[:, a:a + 1]) * mv
            vagg = vmsg.reshape(NB, SLOTS, H).sum(axis=1)
            veco_ref[a, :, :] = (vec_ref[a] + vec3_ref[a] * o1 + vagg)

    full = lambda shape: pl.BlockSpec(shape, lambda b: tuple(0 for _ in shape))
    return pl.pallas_call(
        body,
        grid=(N // NB,),
        in_specs=[pl.BlockSpec((NB, H), lambda b: (b, 0)),
                  pl.BlockSpec((EB, 7 * H), lambda b: (b, 0)),
                  pl.BlockSpec((EB, RBF), lambda b: (b, 0)),
                  pl.BlockSpec((EB, 1), lambda b: (b, 0)),
                  pl.BlockSpec((EB, 1), lambda b: (b, 0)),
                  pl.BlockSpec((EB, 4), lambda b: (b, 0)),
                  pl.BlockSpec((NB, H), lambda b: (b, 0)),
                  pl.BlockSpec((3, NB, H), lambda b: (0, b, 0)),
                  pl.BlockSpec((NB, H), lambda b: (b, 0)),
                  pl.BlockSpec((3, NB, H), lambda b: (0, b, 0)),
                  full((RBF, H)), full((1, H)),
                  full((RBF, 3 * H)), full((1, 3 * H)),
                  full((H, 3 * H)), full((1, 3 * H))],
        out_specs=[pl.BlockSpec((NB, H), lambda b: (b, 0)),
                   pl.BlockSpec((3, NB, H), lambda b: (0, b, 0))],
        out_shape=[jax.ShapeDtypeStruct((N, H), jnp.float32),
                   jax.ShapeDtypeStruct((3, N, H), jnp.float32)],
        interpret=_INTERP,
    )(q, g, f, cvec, mvec, d4, vdot, vec3S, x, vecS,
      dkw, dkb, dvw, dvb, ow, ob)


def _final_ln_kernel(x, w, b):
    NB = 512

    def body(x_ref, w_ref, b_ref, o_ref):
        o_ref[...] = _layer_norm(x_ref[...], w_ref[...], b_ref[...])

    full = lambda shape: pl.BlockSpec(shape, lambda b: tuple(0 for _ in shape))
    return pl.pallas_call(
        body,
        grid=(N // NB,),
        in_specs=[pl.BlockSpec((NB, H), lambda b: (b, 0)),
                  full((1, H)), full((1, H))],
        out_specs=pl.BlockSpec((NB, H), lambda b: (b, 0)),
        out_shape=jax.ShapeDtypeStruct((N, H), jnp.float32),
        interpret=_INTERP,
    )(x, w, b)


def _vperm():
    j = jnp.arange(3 * H)
    p, jj = j // H, j % H
    return 96 * (jj // HDIM) + HDIM * p + (jj % HDIM)


def kernel(z, pos, batch, params):
    pos = pos.astype(jnp.float32)
    z = z.astype(jnp.int32)
    batch = batch.astype(jnp.int32)

    pos8 = jnp.pad(pos, ((0, 0), (0, 5)))
    pos128 = jnp.pad(pos, ((0, 0), (0, 125)))
    posT = pos8.T
    batch_c = batch.reshape(N, 1)
    batch_cT = batch.reshape(1, N)

    nbr, mask = _nbr_kernel(pos8, posT, batch_c, batch_cT)
    src = nbr.reshape(E)
    mask_e = mask.reshape(E, 1)

    embcat = jnp.concatenate(
        [params['embedding'], params['nb_embedding']], axis=1)
    emb = _sc_gather(embcat, z)
    x0 = emb[:, :H]
    xz = emb[:, H:]

    pos_s = _sc_gather(pos128, src).reshape(N, SLOTS, 128)
    means = params['means'].reshape(1, RBF)
    betas = params['betas'].reshape(1, RBF)
    f, d4, cvec = _edge_pre_kernel(pos8, pos_s, means, betas)

    xz_s = _sc_gather(xz, src)
    x = _nbr_embed_kernel(
        f, xz_s, cvec, mask_e, x0,
        params['nb_dist_w'], params['nb_dist_b'].reshape(1, H),
        params['nb_comb_w'], params['nb_comb_b'].reshape(1, H))

    vecS = jnp.zeros((3, N, H), jnp.float32)
    vp = _vperm()
    for lp in params['layers']:
        q, tab, vdot, vec3S = _layer_pre_kernel(
            x, vecS,
            lp['ln_w'].reshape(1, H), lp['ln_b'].reshape(1, H),
            lp['q_w'], lp['q_b'].reshape(1, H),
            lp['k_w'], lp['k_b'].reshape(1, H),
            lp['v_w'][:, vp], lp['v_b'][vp].reshape(1, 3 * H),
            lp['vec_w'])
        g = _sc_gather(tab, src)
        x, vecS = _layer_edge_kernel(
            q, g, f, cvec, mask_e, d4, vdot, vec3S, x, vecS,
            lp['dk_w'], lp['dk_b'].reshape(1, H),
            lp['dv_w'][:, vp], lp['dv_b'][vp].reshape(1, 3 * H),
            lp['o_w'], lp['o_b'].reshape(1, 3 * H))

    x = _final_ln_kernel(
        x, params['out_ln_w'].reshape(1, H), params['out_ln_b'].reshape(1, H))
    vec = jnp.transpose(vecS, (1, 0, 2))
    return x, vec


# speedup vs baseline: 2.8329x; 2.8329x over previous
import functools

import jax
import jax.numpy as jnp
from jax import lax
from jax.experimental import pallas as pl
from jax.experimental.pallas import tpu as pltpu
from jax.experimental.pallas import tpu_sc as plsc

N = 4096
H = 256
NHEAD = 8
HDIM = 32
RBF = 64
SLOTS = 32
E = N * SLOTS
CUT = 5.0
_INTERP = False


def _sc_gather(table, idx):
    R, D = table.shape
    B = idx.shape[0]
    NC, NW = 2, 32
    RPW = B // NW
    CK = min(128, RPW)
    while 2 * CK * D * 4 > 470_000:
        CK //= 2
    nch = RPW // CK
    mesh = plsc.VectorSubcoreMesh(core_axis_name="c", subcore_axis_name="s")

    @functools.partial(
        pl.kernel, mesh=mesh,
        out_type=jax.ShapeDtypeStruct((B, D), jnp.float32),
        scratch_types=[
            pltpu.VMEM((RPW,), jnp.int32),
            pltpu.VMEM((2, CK, D), jnp.float32),
            pltpu.SemaphoreType.DMA((2,)),
        ],
    )
    def k(table_hbm, idx_hbm, out_hbm, idx_v, rows_v, sem):
        wid = lax.axis_index("s") * NC + lax.axis_index("c")
        base0 = wid * RPW
        pltpu.sync_copy(idx_hbm.at[pl.ds(base0, RPW)], idx_v)

        def start(c, slot):
            pltpu.async_copy(table_hbm.at[idx_v.at[pl.ds(c * CK, CK)]],
                             rows_v.at[slot], sem.at[slot])

        start(0, 0)

        def body(c, carry):
            slot = lax.rem(c, 2)

            @pl.when(c + 1 < nch)
            def _():
                start(c + 1, lax.rem(c + 1, 2))

            pltpu.make_async_copy(
                table_hbm.at[idx_v.at[pl.ds(c * CK, CK)]],
                rows_v.at[slot], sem.at[slot]).wait()
            pltpu.sync_copy(rows_v.at[slot],
                            out_hbm.at[pl.ds(base0 + c * CK, CK)])
            return carry

        lax.fori_loop(0, nch, body, 0)

    return k(table, idx)


def _silu(x):
    return x / (1.0 + jnp.exp(-x))


def _nbr_kernel(pos8, posT, batch_c, batch_cT):
    RB = 256
    W = 1024
    INF = float(jnp.inf)

    def body(pos_ref, posT_ref, bc_ref, bcT_ref, nbr_ref, msk_ref):
        b = pl.program_id(0)
        r0 = b * RB
        bcol = bc_ref[...]
        rsel = lax.broadcasted_iota(jnp.int32, (N, 1), 0) == r0
        b0 = jnp.sum(jnp.where(rsel, bcol, 0))
        w = jnp.sum((bcol < b0).astype(jnp.int32))
        w = jnp.minimum((w // 128) * 128, N - W)
        w = pl.multiple_of(w, 128)

        pr = pos_ref[pl.ds(r0, RB), :]
        pcT = posT_ref[:, pl.ds(w, W)]
        sq_r = pr[:, 0:1] ** 2 + pr[:, 1:2] ** 2 + pr[:, 2:3] ** 2
        sq_c = pcT[0:1, :] ** 2 + pcT[1:2, :] ** 2 + pcT[2:3, :] ** 2
        dot = jax.lax.dot_general(
            pr, pcT, (((1,), (0,)), ((), ())),
            precision=jax.lax.Precision.HIGHEST)
        d2 = (sq_r + sq_c) - 2.0 * dot
        d = jnp.sqrt(jnp.maximum(d2, 0.0))

        br = bc_ref[pl.ds(r0, RB), :]
        bcT_w = bcT_ref[:, pl.ds(w, W)]
        colidx = w + lax.broadcasted_iota(jnp.int32, (1, W), 1)
        rowidx = r0 + lax.broadcasted_iota(jnp.int32, (RB, 1), 0)
        valid = (br == bcT_w) & (d < CUT) & (colidx != rowidx)
        dw = jnp.where(valid, d, INF)

        lane = lax.broadcasted_iota(jnp.int32, (1, SLOTS), 1)
        BIG = jnp.int32(2 ** 30)

        def step(t, carry):
            dwc, nbr, msk = carry
            m = jnp.min(dwc, axis=1, keepdims=True)
            am = jnp.min(jnp.where(dwc == m, colidx, BIG),
                         axis=1, keepdims=True)
            has = m < INF
            amc = jnp.where(has, am, 0)
            onec = lane == t
            nbr = jnp.where(onec, amc, nbr)
            msk = jnp.where(onec, has.astype(jnp.float32), msk)
            dwc = jnp.where(colidx == am, INF, dwc)
            return dwc, nbr, msk

        nbr0 = jnp.zeros((RB, SLOTS), jnp.int32)
        msk0 = jnp.zeros((RB, SLOTS), jnp.float32)
        _, nbr, msk = lax.fori_loop(0, SLOTS, step, (dw, nbr0, msk0))
        nbr_ref[...] = nbr
        msk_ref[...] = msk

    full = lambda shape: pl.BlockSpec(shape, lambda b: tuple(0 for _ in shape))
    return pl.pallas_call(
        body,
        grid=(N // RB,),
        in_specs=[full((N, 8)), full((8, N)), full((N, 1)), full((1, N))],
        out_specs=[pl.BlockSpec((RB, SLOTS), lambda b: (b, 0)),
                   pl.BlockSpec((RB, SLOTS), lambda b: (b, 0))],
        out_shape=[jax.ShapeDtypeStruct((N, SLOTS), jnp.int32),
                   jax.ShapeDtypeStruct((N, SLOTS), jnp.float32)],
        interpret=_INTERP,
    )(pos8, posT, batch_c, batch_cT)


def _edge_pre_kernel(pos8, pos_s3, means, betas):
    NB = 128
    EB = NB * SLOTS

    def body(pos_ref, ps_ref, mean_ref, beta_ref, f_ref, d4_ref, c_ref):
        p = pos_ref[...]
        ps = ps_ref[...]
        prep = jnp.broadcast_to(p[:, None, :], (NB, SLOTS, 8))
        prep = prep.reshape(EB, 8)
        psf = ps.reshape(EB, 128)
        evx = psf[:, 0:1] - prep[:, 0:1]
        evy = psf[:, 1:2] - prep[:, 1:2]
        evz = psf[:, 2:3] - prep[:, 2:3]
        wgt = jnp.sqrt(evx * evx + evy * evy + evz * evz)
        wsafe = jnp.where(wgt > 0.0, wgt, 1.0)
        zero = jnp.zeros((EB, 1), jnp.float32)
        d4_ref[...] = jnp.concatenate(
            [evx / wsafe, evy / wsafe, evz / wsafe, zero], axis=1)
        cosc = 0.5 * (jnp.cos(wgt * (jnp.pi / CUT)) + 1.0)
        cosc = cosc * (wgt < CUT).astype(jnp.float32)
        c_ref[...] = cosc
        mu = mean_ref[...]
        be = beta_ref[...]
        t = jnp.exp(-wgt) - mu
        f_ref[...] = cosc * jnp.exp(-be * t * t)

    full = lambda shape: pl.BlockSpec(shape, lambda b: tuple(0 for _ in shape))
    return pl.pallas_call(
        body,
        grid=(N // NB,),
        in_specs=[pl.BlockSpec((NB, 8), lambda b: (b, 0)),
                  pl.BlockSpec((NB, SLOTS, 128), lambda b: (b, 0, 0)),
                  full((1, RBF)), full((1, RBF))],
        out_specs=[pl.BlockSpec((EB, RBF), lambda b: (b, 0)),
                   pl.BlockSpec((EB, 4), lambda b: (b, 0)),
                   pl.BlockSpec((EB, 1), lambda b: (b, 0))],
        out_shape=[jax.ShapeDtypeStruct((E, RBF), jnp.float32),
                   jax.ShapeDtypeStruct((E, 4), jnp.float32),
                   jax.ShapeDtypeStruct((E, 1), jnp.float32)],
        interpret=_INTERP,
    )(pos8, pos_s3, means, betas)


def _nbr_embed_kernel(f, xz_s, cvec, mvec, x0, dw, db, cw, cb):
    NB = 128
    EB = NB * SLOTS

    def body(f_ref, xz_ref, c_ref, m_ref, x0_ref, dw_ref, db_ref,
             cw_ref, cb_ref, out_ref):
        Wm = jnp.dot(f_ref[...], dw_ref[...]) + db_ref[...]
        Wm = Wm * c_ref[...] * m_ref[...]
        msgs = Wm * xz_ref[...]
        agg = msgs.reshape(NB, SLOTS, H).sum(axis=1)
        xcat = jnp.concatenate([x0_ref[...], agg], axis=1)
        out_ref[...] = jnp.dot(xcat, cw_ref[...]) + cb_ref[...]

    full = lambda shape: pl.BlockSpec(shape, lambda b: tuple(0 for _ in shape))
    return pl.pallas_call(
        body,
        grid=(N // NB,),
        in_specs=[pl.BlockSpec((EB, RBF), lambda b: (b, 0)),
                  pl.BlockSpec((EB, H), lambda b: (b, 0)),
                  pl.BlockSpec((EB, 1), lambda b: (b, 0)),
                  pl.BlockSpec((EB, 1), lambda b: (b, 0)),
                  pl.BlockSpec((NB, H), lambda b: (b, 0)),
                  full((RBF, H)), full((1, H)),
                  full((2 * H, H)), full((1, H))],
        out_specs=pl.BlockSpec((NB, H), lambda b: (b, 0)),
        out_shape=jax.ShapeDtypeStruct((N, H), jnp.float32),
        interpret=_INTERP,
    )(f, xz_s, cvec, mvec, x0, dw, db, cw, cb)


def _layer_norm(x, w, b):
    mu = jnp.mean(x, axis=-1, keepdims=True)
    var = jnp.mean((x - mu) ** 2, axis=-1, keepdims=True)
    return (x - mu) / jnp.sqrt(var + 1e-5) * w + b


def _layer_pre_kernel(x, vecS, lnw, lnb, qw, qb, kw, kb, vw, vb, vecw):
    NB = 256

    def body(x_ref, vec_ref, lnw_ref, lnb_ref, qw_ref, qb_ref, kw_ref,
             kb_ref, vw_ref, vb_ref, vecw_ref,
             q_ref, tab_ref, vdot_ref, vec3_ref):
        xn = _layer_norm(x_ref[...], lnw_ref[...], lnb_ref[...])
        q = jnp.dot(xn, qw_ref[...]) + qb_ref[...]
        k = jnp.dot(xn, kw_ref[...]) + kb_ref[...]
        v = jnp.dot(xn, vw_ref[...]) + vb_ref[...]
        q_ref[...] = q
        vec = vec_ref[...]
        vdot = jnp.zeros((NB, H), jnp.float32)
        vecw_v = vecw_ref[...]
        for a in range(3):
            vp = jnp.dot(vec[a], vecw_v)
            vdot = vdot + vp[:, :H] * vp[:, H:2 * H]
            vec3_ref[a, :, :] = vp[:, 2 * H:]
        vdot_ref[...] = vdot
        tab_ref[...] = jnp.concatenate(
            [k, v, vec[0], vec[1], vec[2]], axis=1).astype(jnp.bfloat16)

    full = lambda shape: pl.BlockSpec(shape, lambda b: tuple(0 for _ in shape))
    return pl.pallas_call(
        body,
        grid=(N // NB,),
        in_specs=[pl.BlockSpec((NB, H), lambda b: (b, 0)),
                  pl.BlockSpec((3, NB, H), lambda b: (0, b, 0)),
                  full((1, H)), full((1, H)),
                  full((H, H)), full((1, H)),
                  full((H, H)), full((1, H)),
                  full((H, 3 * H)), full((1, 3 * H)),
                  full((H, 3 * H))],
        out_specs=[pl.BlockSpec((NB, H), lambda b: (b, 0)),
                   pl.BlockSpec((NB, 7 * H), lambda b: (b, 0)),
                   pl.BlockSpec((NB, H), lambda b: (b, 0)),
                   pl.BlockSpec((3, NB, H), lambda b: (0, b, 0))],
        out_shape=[jax.ShapeDtypeStruct((N, H), jnp.float32),
                   jax.ShapeDtypeStruct((N, 7 * H), jnp.bfloat16),
                   jax.ShapeDtypeStruct((N, H), jnp.float32),
                   jax.ShapeDtypeStruct((3, N, H), jnp.float32)],
        interpret=_INTERP,
    )(x, vecS, lnw, lnb, qw, qb, kw, kb, vw, vb, vecw)


def _win_starts_kernel(batch_c, batch_cT):
    NBLK = N // 32
    WROWS = 768

    def body(bc_ref, bcT_ref, ws_ref):
        firsts = bc_ref[...].reshape(NBLK, 32)[:, 0:1]
        cmp = (bcT_ref[...] < firsts).astype(jnp.int32)
        cnt = cmp.sum(axis=1, keepdims=True)
        ws = jnp.minimum((cnt // 128) * 128, N - WROWS)
        ws_ref[...] = ws // 128

    full = lambda shape: pl.BlockSpec(shape, lambda: tuple(0 for _ in shape))
    return pl.pallas_call(
        body,
        in_specs=[full((N, 1)), full((1, N))],
        out_specs=full((NBLK, 1)),
        out_shape=jax.ShapeDtypeStruct((NBLK, 1), jnp.int32),
        interpret=_INTERP,
    )(batch_c, batch_cT)


def _layer_edge_kernel(ws, q, tab, src_e, f, cvec, mvec, d4, vdot, vec3S,
                       x, vecS, dkw, dkb, dvw, dvb, ow, ob):
    NB = 32
    EB = NB * SLOTS
    NWIN = 6

    def body(ws_ref, q_ref, t0_ref, t1_ref, t2_ref, t3_ref, t4_ref, t5_ref,
             src_ref, f_ref, c_ref, m_ref, d4_ref, vdot_ref,
             vec3_ref, x_ref, vec_ref, dkw_ref, dkb_ref, dvw_ref,
             dvb_ref, ow_ref, ob_ref, xo_ref, veco_ref):
        fv = f_ref[...]
        dk = _silu(jnp.dot(fv, dkw_ref[...]) + dkb_ref[...])
        dv = _silu(jnp.dot(fv, dvw_ref[...]) + dvb_ref[...])
        base = ws_ref[pl.program_id(0)] * 128
        srcv = src_ref[...]
        lane = lax.broadcasted_iota(jnp.int32, (1, 128), 1)
        g = jnp.zeros((EB, 7 * H), jnp.float32)
        for t_ref in (t0_ref, t1_ref, t2_ref, t3_ref, t4_ref, t5_ref):
            oh = (srcv == base + lane).astype(jnp.bfloat16)
            g = g + jax.lax.dot_general(
                oh, t_ref[...], (((1,), (0,)), ((), ())),
                preferred_element_type=jnp.float32)
            base = base + 128
        qv = q_ref[...]
        qrep = jnp.broadcast_to(qv[:, None, :], (NB, SLOTS, H)).reshape(EB, H)
        prod = g[:, :H] * dk * qrep
        hsel = (lax.broadcasted_iota(jnp.int32, (H, NHEAD), 0) // HDIM ==
                lax.broadcasted_iota(jnp.int32, (H, NHEAD), 1)
                ).astype(jnp.float32)
        attn = jnp.dot(prod, hsel)
        attn = _silu(attn) * c_ref[...] * m_ref[...]
        attn_exp = jnp.dot(attn, hsel.T)
        vj = g[:, H:4 * H] * dv
        xmsg = vj[:, :H] * attn_exp
        x_agg = xmsg.reshape(NB, SLOTS, H).sum(axis=1)
        o = jnp.dot(x_agg, ow_ref[...]) + ob_ref[...]
        o1, o2, o3 = o[:, :H], o[:, H:2 * H], o[:, 2 * H:]
        xo_ref[...] = x_ref[...] + vdot_ref[...] * o2 + o3
        v1m = vj[:, H:2 * H]
        v2m = vj[:, 2 * H:]
        mv = m_ref[...]
        d4v = d4_ref[...]
        for a in range(3):
            vmsg = (g[:, (4 + a) * H:(5 + a) * H] * v1m
                    + v2m * d4v[:, a:a + 1]) * mv
            vagg = vmsg.reshape(NB, SLOTS, H).sum(axis=1)
            veco_ref[a, :, :] = (vec_ref[a] + vec3_ref[a] * o1 + vagg)

    full = lambda shape: pl.BlockSpec(shape, lambda b, w: tuple(0 for _ in shape))
    win = lambda k: pl.BlockSpec((128, 7 * H), lambda b, w: (w[b] + k, 0))
    grid_spec = pltpu.PrefetchScalarGridSpec(
        num_scalar_prefetch=1,
        grid=(N // NB,),
        in_specs=[pl.BlockSpec((NB, H), lambda b, w: (b, 0)),
                  win(0), win(1), win(2), win(3), win(4), win(5),
                  pl.BlockSpec((EB, 1), lambda b, w: (b, 0)),
                  pl.BlockSpec((EB, RBF), lambda b, w: (b, 0)),
                  pl.BlockSpec((EB, 1), lambda b, w: (b, 0)),
                  pl.BlockSpec((EB, 1), lambda b, w: (b, 0)),
                  pl.BlockSpec((EB, 4), lambda b, w: (b, 0)),
                  pl.BlockSpec((NB, H), lambda b, w: (b, 0)),
                  pl.BlockSpec((3, NB, H), lambda b, w: (0, b, 0)),
                  pl.BlockSpec((NB, H), lambda b, w: (b, 0)),
                  pl.BlockSpec((3, NB, H), lambda b, w: (0, b, 0)),
                  full((RBF, H)), full((1, H)),
                  full((RBF, 3 * H)), full((1, 3 * H)),
                  full((H, 3 * H)), full((1, 3 * H))],
        out_specs=[pl.BlockSpec((NB, H), lambda b, w: (b, 0)),
                   pl.BlockSpec((3, NB, H), lambda b, w: (0, b, 0))],
    )
    return pl.pallas_call(
        body,
        grid_spec=grid_spec,
        out_shape=[jax.ShapeDtypeStruct((N, H), jnp.float32),
                   jax.ShapeDtypeStruct((3, N, H), jnp.float32)],
        interpret=_INTERP,
    )(ws, q, tab, tab, tab, tab, tab, tab, src_e, f, cvec, mvec, d4,
      vdot, vec3S, x, vecS, dkw, dkb, dvw, dvb, ow, ob)


def _final_ln_kernel(x, w, b):
    NB = 512

    def body(x_ref, w_ref, b_ref, o_ref):
        o_ref[...] = _layer_norm(x_ref[...], w_ref[...], b_ref[...])

    full = lambda shape: pl.BlockSpec(shape, lambda b: tuple(0 for _ in shape))
    return pl.pallas_call(
        body,
        grid=(N // NB,),
        in_specs=[pl.BlockSpec((NB, H), lambda b: (b, 0)),
                  full((1, H)), full((1, H))],
        out_specs=pl.BlockSpec((NB, H), lambda b: (b, 0)),
        out_shape=jax.ShapeDtypeStruct((N, H), jnp.float32),
        interpret=_INTERP,
    )(x, w, b)


def _vperm():
    j = jnp.arange(3 * H)
    p, jj = j // H, j % H
    return 96 * (jj // HDIM) + HDIM * p + (jj % HDIM)


def kernel(z, pos, batch, params):
    pos = pos.astype(jnp.float32)
    z = z.astype(jnp.int32)
    batch = batch.astype(jnp.int32)

    pos8 = jnp.pad(pos, ((0, 0), (0, 5)))
    pos128 = jnp.pad(pos, ((0, 0), (0, 125)))
    posT = pos8.T
    batch_c = batch.reshape(N, 1)
    batch_cT = batch.reshape(1, N)

    nbr, mask = _nbr_kernel(pos8, posT, batch_c, batch_cT)
    src = nbr.reshape(E)
    src_e = nbr.reshape(E, 1)
    mask_e = mask.reshape(E, 1)
    ws = _win_starts_kernel(batch_c, batch_cT).reshape(N // 32)

    embcat = jnp.concatenate(
        [params['embedding'], params['nb_embedding']], axis=1)
    emb = _sc_gather(embcat, z)
    x0 = emb[:, :H]
    xz = emb[:, H:]

    pos_s = _sc_gather(pos128, src).reshape(N, SLOTS, 128)
    means = params['means'].reshape(1, RBF)
    betas = params['betas'].reshape(1, RBF)
    f, d4, cvec = _edge_pre_kernel(pos8, pos_s, means, betas)

    xz_s = _sc_gather(xz, src)
    x = _nbr_embed_kernel(
        f, xz_s, cvec, mask_e, x0,
        params['nb_dist_w'], params['nb_dist_b'].reshape(1, H),
        params['nb_comb_w'], params['nb_comb_b'].reshape(1, H))

    vecS = jnp.zeros((3, N, H), jnp.float32)
    vp = _vperm()
    for lp in params['layers']:
        q, tab, vdot, vec3S = _layer_pre_kernel(
            x, vecS,
            lp['ln_w'].reshape(1, H), lp['ln_b'].reshape(1, H),
            lp['q_w'], lp['q_b'].reshape(1, H),
            lp['k_w'], lp['k_b'].reshape(1, H),
            lp['v_w'][:, vp], lp['v_b'][vp].reshape(1, 3 * H),
            lp['vec_w'])
        x, vecS = _layer_edge_kernel(
            ws, q, tab, src_e, f, cvec, mask_e, d4, vdot, vec3S, x, vecS,
            lp['dk_w'], lp['dk_b'].reshape(1, H),
            lp['dv_w'][:, vp], lp['dv_b'][vp].reshape(1, 3 * H),
            lp['o_w'], lp['o_b'].reshape(1, 3 * H))

    x = _final_ln_kernel(
        x, params['out_ln_w'].reshape(1, H), params['out_ln_b'].reshape(1, H))
    vec = jnp.transpose(vecS, (1, 0, 2))
    return x, vec


# speedup vs baseline: 4.2506x; 1.5005x over previous
import functools

import jax
import jax.numpy as jnp
from jax import lax
from jax.experimental import pallas as pl
from jax.experimental.pallas import tpu as pltpu
from jax.experimental.pallas import tpu_sc as plsc

N = 4096
H = 256
NHEAD = 8
HDIM = 32
RBF = 64
SLOTS = 32
E = N * SLOTS
CUT = 5.0
_INTERP = False


def _sc_gather(table, idx):
    R, D = table.shape
    B = idx.shape[0]
    NC, NW = 2, 32
    RPW = B // NW
    CK = min(128, RPW)
    while 2 * CK * D * 4 > 470_000:
        CK //= 2
    nch = RPW // CK
    mesh = plsc.VectorSubcoreMesh(core_axis_name="c", subcore_axis_name="s")

    @functools.partial(
        pl.kernel, mesh=mesh,
        out_type=jax.ShapeDtypeStruct((B, D), jnp.float32),
        scratch_types=[
            pltpu.VMEM((RPW,), jnp.int32),
            pltpu.VMEM((2, CK, D), jnp.float32),
            pltpu.SemaphoreType.DMA((2,)),
        ],
    )
    def k(table_hbm, idx_hbm, out_hbm, idx_v, rows_v, sem):
        wid = lax.axis_index("s") * NC + lax.axis_index("c")
        base0 = wid * RPW
        pltpu.sync_copy(idx_hbm.at[pl.ds(base0, RPW)], idx_v)

        def start(c, slot):
            pltpu.async_copy(table_hbm.at[idx_v.at[pl.ds(c * CK, CK)]],
                             rows_v.at[slot], sem.at[slot])

        start(0, 0)

        def body(c, carry):
            slot = lax.rem(c, 2)

            @pl.when(c + 1 < nch)
            def _():
                start(c + 1, lax.rem(c + 1, 2))

            pltpu.make_async_copy(
                table_hbm.at[idx_v.at[pl.ds(c * CK, CK)]],
                rows_v.at[slot], sem.at[slot]).wait()
            pltpu.sync_copy(rows_v.at[slot],
                            out_hbm.at[pl.ds(base0 + c * CK, CK)])
            return carry

        lax.fori_loop(0, nch, body, 0)

    return k(table, idx)


def _silu(x):
    return x / (1.0 + jnp.exp(-x))


def _nbr_kernel(pos8, posT, batch_c, batch_cT):
    RB = 256
    W = 1024
    INF = float(jnp.inf)

    def body(pos_ref, posT_ref, bc_ref, bcT_ref, nbr_ref, msk_ref):
        b = pl.program_id(0)
        r0 = b * RB
        bcol = bc_ref[...]
        rsel = lax.broadcasted_iota(jnp.int32, (N, 1), 0) == r0
        b0 = jnp.sum(jnp.where(rsel, bcol, 0))
        w = jnp.sum((bcol < b0).astype(jnp.int32))
        w = jnp.minimum((w // 128) * 128, N - W)
        w = pl.multiple_of(w, 128)

        pr = pos_ref[pl.ds(r0, RB), :]
        pcT = posT_ref[:, pl.ds(w, W)]
        sq_r = pr[:, 0:1] ** 2 + pr[:, 1:2] ** 2 + pr[:, 2:3] ** 2
        sq_c = pcT[0:1, :] ** 2 + pcT[1:2, :] ** 2 + pcT[2:3, :] ** 2
        dot = jax.lax.dot_general(
            pr, pcT, (((1,), (0,)), ((), ())),
            precision=jax.lax.Precision.HIGHEST)
        d2 = (sq_r + sq_c) - 2.0 * dot
        d = jnp.sqrt(jnp.maximum(d2, 0.0))

        br = bc_ref[pl.ds(r0, RB), :]
        bcT_w = bcT_ref[:, pl.ds(w, W)]
        colidx = w + lax.broadcasted_iota(jnp.int32, (1, W), 1)
        rowidx = r0 + lax.broadcasted_iota(jnp.int32, (RB, 1), 0)
        valid = (br == bcT_w) & (d < CUT) & (colidx != rowidx)
        dw = jnp.where(valid, d, INF)

        lane = lax.broadcasted_iota(jnp.int32, (1, SLOTS), 1)
        BIG = jnp.int32(2 ** 30)

        def step(t, carry):
            dwc, nbr, msk = carry
            m = jnp.min(dwc, axis=1, keepdims=True)
            am = jnp.min(jnp.where(dwc == m, colidx, BIG),
                         axis=1, keepdims=True)
            has = m < INF
            amc = jnp.where(has, am, 0)
            onec = lane == t
            nbr = jnp.where(onec, amc, nbr)
            msk = jnp.where(onec, has.astype(jnp.float32), msk)
            dwc = jnp.where(colidx == am, INF, dwc)
            return dwc, nbr, msk

        nbr0 = jnp.zeros((RB, SLOTS), jnp.int32)
        msk0 = jnp.zeros((RB, SLOTS), jnp.float32)
        _, nbr, msk = lax.fori_loop(0, SLOTS, step, (dw, nbr0, msk0))
        nbr_ref[...] = nbr
        msk_ref[...] = msk

    full = lambda shape: pl.BlockSpec(shape, lambda b: tuple(0 for _ in shape))
    return pl.pallas_call(
        body,
        grid=(N // RB,),
        in_specs=[full((N, 8)), full((8, N)), full((N, 1)), full((1, N))],
        out_specs=[pl.BlockSpec((RB, SLOTS), lambda b: (b, 0)),
                   pl.BlockSpec((RB, SLOTS), lambda b: (b, 0))],
        out_shape=[jax.ShapeDtypeStruct((N, SLOTS), jnp.int32),
                   jax.ShapeDtypeStruct((N, SLOTS), jnp.float32)],
        interpret=_INTERP,
    )(pos8, posT, batch_c, batch_cT)


def _edge_pre_kernel(ws, pos8, posw, src_e, means, betas):
    NB = 32
    EB = NB * SLOTS

    def body(ws_ref, pos_ref, w0, w1, w2, w3, w4, w5, src_ref,
             mean_ref, beta_ref, f_ref, d4_ref, c_ref):
        p = pos_ref[...]
        base = ws_ref[pl.program_id(0)] * 128
        srcv = src_ref[...]
        lane = lax.broadcasted_iota(jnp.int32, (1, 128), 1)
        psf = jnp.zeros((EB, 128), jnp.float32)
        for w_ref in (w0, w1, w2, w3, w4, w5):
            oh = (srcv == base + lane).astype(jnp.float32)
            psf = psf + jax.lax.dot_general(
                oh, w_ref[...], (((1,), (0,)), ((), ())),
                precision=jax.lax.Precision.HIGHEST)
            base = base + 128
        prep = jnp.broadcast_to(p[:, None, :], (NB, SLOTS, 8))
        prep = prep.reshape(EB, 8)
        evx = psf[:, 0:1] - prep[:, 0:1]
        evy = psf[:, 1:2] - prep[:, 1:2]
        evz = psf[:, 2:3] - prep[:, 2:3]
        wgt = jnp.sqrt(evx * evx + evy * evy + evz * evz)
        wsafe = jnp.where(wgt > 0.0, wgt, 1.0)
        zero = jnp.zeros((EB, 1), jnp.float32)
        d4_ref[...] = jnp.concatenate(
            [evx / wsafe, evy / wsafe, evz / wsafe, zero], axis=1)
        cosc = 0.5 * (jnp.cos(wgt * (jnp.pi / CUT)) + 1.0)
        cosc = cosc * (wgt < CUT).astype(jnp.float32)
        c_ref[...] = cosc
        mu = mean_ref[...]
        be = beta_ref[...]
        t = jnp.exp(-wgt) - mu
        f_ref[...] = cosc * jnp.exp(-be * t * t)

    full = lambda shape: pl.BlockSpec(shape, lambda b, w: tuple(0 for _ in shape))
    win = lambda k: pl.BlockSpec((128, 128), lambda b, w: (w[b] + k, 0))
    grid_spec = pltpu.PrefetchScalarGridSpec(
        num_scalar_prefetch=1,
        grid=(N // NB,),
        in_specs=[pl.BlockSpec((NB, 8), lambda b, w: (b, 0)),
                  win(0), win(1), win(2), win(3), win(4), win(5),
                  pl.BlockSpec((EB, 1), lambda b, w: (b, 0)),
                  full((1, RBF)), full((1, RBF))],
        out_specs=[pl.BlockSpec((EB, RBF), lambda b, w: (b, 0)),
                   pl.BlockSpec((EB, 4), lambda b, w: (b, 0)),
                   pl.BlockSpec((EB, 1), lambda b, w: (b, 0))],
    )
    return pl.pallas_call(
        body,
        grid_spec=grid_spec,
        out_shape=[jax.ShapeDtypeStruct((E, RBF), jnp.float32),
                   jax.ShapeDtypeStruct((E, 4), jnp.float32),
                   jax.ShapeDtypeStruct((E, 1), jnp.float32)],
        interpret=_INTERP,
    )(ws, pos8, posw, posw, posw, posw, posw, posw, src_e, means, betas)


def _nbr_embed_kernel(ws, f, xz, src_e, cvec, mvec, x0, dw, db, cw, cb):
    NB = 32
    EB = NB * SLOTS

    def body(ws_ref, f_ref, w0, w1, w2, w3, w4, w5, src_ref, c_ref, m_ref,
             x0_ref, dw_ref, db_ref, cw_ref, cb_ref, out_ref):
        base = ws_ref[pl.program_id(0)] * 128
        srcv = src_ref[...]
        lane = lax.broadcasted_iota(jnp.int32, (1, 128), 1)
        xz_s = jnp.zeros((EB, H), jnp.float32)
        for w_ref in (w0, w1, w2, w3, w4, w5):
            oh = (srcv == base + lane).astype(jnp.float32)
            xz_s = xz_s + jax.lax.dot_general(
                oh, w_ref[...], (((1,), (0,)), ((), ())),
                precision=jax.lax.Precision.HIGHEST)
            base = base + 128
        Wm = jnp.dot(f_ref[...], dw_ref[...]) + db_ref[...]
        Wm = Wm * c_ref[...] * m_ref[...]
        msgs = Wm * xz_s
        agg = msgs.reshape(NB, SLOTS, H).sum(axis=1)
        xcat = jnp.concatenate([x0_ref[...], agg], axis=1)
        out_ref[...] = jnp.dot(xcat, cw_ref[...]) + cb_ref[...]

    full = lambda shape: pl.BlockSpec(shape, lambda b, w: tuple(0 for _ in shape))
    win = lambda k: pl.BlockSpec((128, H), lambda b, w: (w[b] + k, 0))
    grid_spec = pltpu.PrefetchScalarGridSpec(
        num_scalar_prefetch=1,
        grid=(N // NB,),
        in_specs=[pl.BlockSpec((EB, RBF), lambda b, w: (b, 0)),
                  win(0), win(1), win(2), win(3), win(4), win(5),
                  pl.BlockSpec((EB, 1), lambda b, w: (b, 0)),
                  pl.BlockSpec((EB, 1), lambda b, w: (b, 0)),
                  pl.BlockSpec((EB, 1), lambda b, w: (b, 0)),
                  pl.BlockSpec((NB, H), lambda b, w: (b, 0)),
                  full((RBF, H)), full((1, H)),
                  full((2 * H, H)), full((1, H))],
        out_specs=pl.BlockSpec((NB, H), lambda b, w: (b, 0)),
    )
    return pl.pallas_call(
        body,
        grid_spec=grid_spec,
        out_shape=jax.ShapeDtypeStruct((N, H), jnp.float32),
        interpret=_INTERP,
    )(ws, f, xz, xz, xz, xz, xz, xz, src_e, cvec, mvec, x0, dw, db, cw, cb)


def _layer_norm(x, w, b):
    mu = jnp.mean(x, axis=-1, keepdims=True)
    var = jnp.mean((x - mu) ** 2, axis=-1, keepdims=True)
    return (x - mu) / jnp.sqrt(var + 1e-5) * w + b


def _layer_pre_kernel(x, vecS, lnw, lnb, qw, qb, kw, kb, vw, vb, vecw):
    NB = 256

    def body(x_ref, vec_ref, lnw_ref, lnb_ref, qw_ref, qb_ref, kw_ref,
             kb_ref, vw_ref, vb_ref, vecw_ref,
             q_ref, tab_ref, vdot_ref, vec3_ref):
        xn = _layer_norm(x_ref[...], lnw_ref[...], lnb_ref[...])
        q = jnp.dot(xn, qw_ref[...]) + qb_ref[...]
        k = jnp.dot(xn, kw_ref[...]) + kb_ref[...]
        v = jnp.dot(xn, vw_ref[...]) + vb_ref[...]
        q_ref[...] = q
        vec = vec_ref[...]
        vdot = jnp.zeros((NB, H), jnp.float32)
        vecw_v = vecw_ref[...]
        for a in range(3):
            vp = jnp.dot(vec[a], vecw_v)
            vdot = vdot + vp[:, :H] * vp[:, H:2 * H]
            vec3_ref[a, :, :] = vp[:, 2 * H:]
        vdot_ref[...] = vdot
        tab_ref[...] = jnp.concatenate(
            [k, v, vec[0], vec[1], vec[2]], axis=1).astype(jnp.bfloat16)

    full = lambda shape: pl.BlockSpec(shape, lambda b: tuple(0 for _ in shape))
    return pl.pallas_call(
        body,
        grid=(N // NB,),
        in_specs=[pl.BlockSpec((NB, H), lambda b: (b, 0)),
                  pl.BlockSpec((3, NB, H), lambda b: (0, b, 0)),
                  full((1, H)), full((1, H)),
                  full((H, H)), full((1, H)),
                  full((H, H)), full((1, H)),
                  full((H, 3 * H)), full((1, 3 * H)),
                  full((H, 3 * H))],
        out_specs=[pl.BlockSpec((NB, H), lambda b: (b, 0)),
                   pl.BlockSpec((NB, 7 * H), lambda b: (b, 0)),
                   pl.BlockSpec((NB, H), lambda b: (b, 0)),
                   pl.BlockSpec((3, NB, H), lambda b: (0, b, 0))],
        out_shape=[jax.ShapeDtypeStruct((N, H), jnp.float32),
                   jax.ShapeDtypeStruct((N, 7 * H), jnp.bfloat16),
                   jax.ShapeDtypeStruct((N, H), jnp.float32),
                   jax.ShapeDtypeStruct((3, N, H), jnp.float32)],
        interpret=_INTERP,
    )(x, vecS, lnw, lnb, qw, qb, kw, kb, vw, vb, vecw)


def _win_starts_kernel(batch32, batch_cT):
    NBLK = N // 32
    WROWS = 768

    def body(b32_ref, bcT_ref, ws_ref):
        firsts = b32_ref[:, 0:1]
        cmp = (bcT_ref[...] < firsts).astype(jnp.int32)
        cnt = cmp.sum(axis=1, keepdims=True)
        ws = jnp.minimum((cnt // 128) * 128, N - WROWS)
        ws_ref[...] = ws // 128

    full = lambda shape: pl.BlockSpec(shape, lambda: tuple(0 for _ in shape))
    return pl.pallas_call(
        body,
        in_specs=[full((NBLK, 32)), full((1, N))],
        out_specs=full((NBLK, 1)),
        out_shape=jax.ShapeDtypeStruct((NBLK, 1), jnp.int32),
        interpret=_INTERP,
    )(batch32, batch_cT)


def _layer_edge_kernel(ws, q, tab, src_e, f, cvec, mvec, d4, vdot, vec3S,
                       x, vecS, dkw, dkb, dvw, dvb, ow, ob):
    NB = 32
    EB = NB * SLOTS
    NWIN = 6

    def body(ws_ref, q_ref, t0_ref, t1_ref, t2_ref, t3_ref, t4_ref, t5_ref,
             src_ref, f_ref, c_ref, m_ref, d4_ref, vdot_ref,
             vec3_ref, x_ref, vec_ref, dkw_ref, dkb_ref, dvw_ref,
             dvb_ref, ow_ref, ob_ref, xo_ref, veco_ref):
        fv = f_ref[...]
        dk = _silu(jnp.dot(fv, dkw_ref[...]) + dkb_ref[...])
        dv = _silu(jnp.dot(fv, dvw_ref[...]) + dvb_ref[...])
        base = ws_ref[pl.program_id(0)] * 128
        srcv = src_ref[...]
        lane = lax.broadcasted_iota(jnp.int32, (1, 128), 1)
        g = jnp.zeros((EB, 7 * H), jnp.float32)
        for t_ref in (t0_ref, t1_ref, t2_ref, t3_ref, t4_ref, t5_ref):
            oh = (srcv == base + lane).astype(jnp.bfloat16)
            g = g + jax.lax.dot_general(
                oh, t_ref[...], (((1,), (0,)), ((), ())),
                preferred_element_type=jnp.float32)
            base = base + 128
        qv = q_ref[...]
        qrep = jnp.broadcast_to(qv[:, None, :], (NB, SLOTS, H)).reshape(EB, H)
        prod = g[:, :H] * dk * qrep
        hsel = (lax.broadcasted_iota(jnp.int32, (H, NHEAD), 0) // HDIM ==
                lax.broadcasted_iota(jnp.int32, (H, NHEAD), 1)
                ).astype(jnp.float32)
        attn = jnp.dot(prod, hsel)
        attn = _silu(attn) * c_ref[...] * m_ref[...]
        attn_exp = jnp.dot(attn, hsel.T)
        vj = g[:, H:4 * H] * dv
        xmsg = vj[:, :H] * attn_exp
        x_agg = xmsg.reshape(NB, SLOTS, H).sum(axis=1)
        o = jnp.dot(x_agg, ow_ref[...]) + ob_ref[...]
        o1, o2, o3 = o[:, :H], o[:, H:2 * H], o[:, 2 * H:]
        xo_ref[...] = x_ref[...] + vdot_ref[...] * o2 + o3
        v1m = vj[:, H:2 * H]
        v2m = vj[:, 2 * H:]
        mv = m_ref[...]
        d4v = d4_ref[...]
        for a in range(3):
            vmsg = (g[:, (4 + a) * H:(5 + a) * H] * v1m
                    + v2m * d4v[:, a:a + 1]) * mv
            vagg = vmsg.reshape(NB, SLOTS, H).sum(axis=1)
            veco_ref[a, :, :] = (vec_ref[a] + vec3_ref[a] * o1 + vagg)

    full = lambda shape: pl.BlockSpec(shape, lambda b, w: tuple(0 for _ in shape))
    win = lambda k: pl.BlockSpec((128, 7 * H), lambda b, w: (w[b] + k, 0))
    grid_spec = pltpu.PrefetchScalarGridSpec(
        num_scalar_prefetch=1,
        grid=(N // NB,),
        in_specs=[pl.BlockSpec((NB, H), lambda b, w: (b, 0)),
                  win(0), win(1), win(2), win(3), win(4), win(5),
                  pl.BlockSpec((EB, 1), lambda b, w: (b, 0)),
                  pl.BlockSpec((EB, RBF), lambda b, w: (b, 0)),
                  pl.BlockSpec((EB, 1), lambda b, w: (b, 0)),
                  pl.BlockSpec((EB, 1), lambda b, w: (b, 0)),
                  pl.BlockSpec((EB, 4), lambda b, w: (b, 0)),
                  pl.BlockSpec((NB, H), lambda b, w: (b, 0)),
                  pl.BlockSpec((3, NB, H), lambda b, w: (0, b, 0)),
                  pl.BlockSpec((NB, H), lambda b, w: (b, 0)),
                  pl.BlockSpec((3, NB, H), lambda b, w: (0, b, 0)),
                  full((RBF, H)), full((1, H)),
                  full((RBF, 3 * H)), full((1, 3 * H)),
                  full((H, 3 * H)), full((1, 3 * H))],
        out_specs=[pl.BlockSpec((NB, H), lambda b, w: (b, 0)),
                   pl.BlockSpec((3, NB, H), lambda b, w: (0, b, 0))],
    )
    return pl.pallas_call(
        body,
        grid_spec=grid_spec,
        out_shape=[jax.ShapeDtypeStruct((N, H), jnp.float32),
                   jax.ShapeDtypeStruct((3, N, H), jnp.float32)],
        interpret=_INTERP,
    )(ws, q, tab, tab, tab, tab, tab, tab, src_e, f, cvec, mvec, d4,
      vdot, vec3S, x, vecS, dkw, dkb, dvw, dvb, ow, ob)


def _final_ln_kernel(x, w, b):
    NB = 512

    def body(x_ref, w_ref, b_ref, o_ref):
        o_ref[...] = _layer_norm(x_ref[...], w_ref[...], b_ref[...])

    full = lambda shape: pl.BlockSpec(shape, lambda b: tuple(0 for _ in shape))
    return pl.pallas_call(
        body,
        grid=(N // NB,),
        in_specs=[pl.BlockSpec((NB, H), lambda b: (b, 0)),
                  full((1, H)), full((1, H))],
        out_specs=pl.BlockSpec((NB, H), lambda b: (b, 0)),
        out_shape=jax.ShapeDtypeStruct((N, H), jnp.float32),
        interpret=_INTERP,
    )(x, w, b)


def _vperm():
    j = jnp.arange(3 * H)
    p, jj = j // H, j % H
    return 96 * (jj // HDIM) + HDIM * p + (jj % HDIM)


def kernel(z, pos, batch, params):
    pos = pos.astype(jnp.float32)
    z = z.astype(jnp.int32)
    batch = batch.astype(jnp.int32)

    pos8 = jnp.pad(pos, ((0, 0), (0, 5)))
    posT = pos8.T
    batch_c = batch.reshape(N, 1)
    batch_cT = batch.reshape(1, N)

    nbr, mask = _nbr_kernel(pos8, posT, batch_c, batch_cT)
    src = nbr.reshape(E)
    src_e = nbr.reshape(E, 1)
    mask_e = mask.reshape(E, 1)
    ws = _win_starts_kernel(batch.reshape(N // 32, 32), batch_cT).reshape(N // 32)

    embcat = jnp.concatenate(
        [params['embedding'], params['nb_embedding']], axis=1)
    emb = _sc_gather(embcat, z)
    x0 = emb[:, :H]
    xz = emb[:, H:]

    means = params['means'].reshape(1, RBF)
    betas = params['betas'].reshape(1, RBF)
    posw = jnp.pad(pos, ((0, 0), (0, 125)))
    f, d4, cvec = _edge_pre_kernel(ws, pos8, posw, src_e, means, betas)

    x = _nbr_embed_kernel(
        ws, f, xz, src_e, cvec, mask_e, x0,
        params['nb_dist_w'], params['nb_dist_b'].reshape(1, H),
        params['nb_comb_w'], params['nb_comb_b'].reshape(1, H))

    vecS = jnp.zeros((3, N, H), jnp.float32)
    vp = _vperm()
    for lp in params['layers']:
        q, tab, vdot, vec3S = _layer_pre_kernel(
            x, vecS,
            lp['ln_w'].reshape(1, H), lp['ln_b'].reshape(1, H),
            lp['q_w'], lp['q_b'].reshape(1, H),
            lp['k_w'], lp['k_b'].reshape(1, H),
            lp['v_w'][:, vp], lp['v_b'][vp].reshape(1, 3 * H),
            lp['vec_w'])
        x, vecS = _layer_edge_kernel(
            ws, q, tab, src_e, f, cvec, mask_e, d4, vdot, vec3S, x, vecS,
            lp['dk_w'], lp['dk_b'].reshape(1, H),
            lp['dv_w'][:, vp], lp['dv_b'][vp].reshape(1, 3 * H),
            lp['o_w'], lp['o_b'].reshape(1, 3 * H))

    x = _final_ln_kernel(
        x, params['out_ln_w'].reshape(1, H), params['out_ln_b'].reshape(1, H))
    vec = jnp.transpose(vecS, (1, 0, 2))
    return x, vec


# speedup vs baseline: 4.7593x; 1.1197x over previous
import functools

import jax
import jax.numpy as jnp
from jax import lax
from jax.experimental import pallas as pl
from jax.experimental.pallas import tpu as pltpu
from jax.experimental.pallas import tpu_sc as plsc

N = 4096
H = 256
NHEAD = 8
HDIM = 32
RBF = 64
SLOTS = 32
E = N * SLOTS
CUT = 5.0
_INTERP = False


def _sc_gather(table, idx):
    R, D = table.shape
    B = idx.shape[0]
    NC, NW = 2, 32
    RPW = B // NW
    CK = min(128, RPW)
    while 2 * CK * D * 4 > 470_000:
        CK //= 2
    nch = RPW // CK
    mesh = plsc.VectorSubcoreMesh(core_axis_name="c", subcore_axis_name="s")

    @functools.partial(
        pl.kernel, mesh=mesh,
        out_type=jax.ShapeDtypeStruct((B, D), jnp.float32),
        scratch_types=[
            pltpu.VMEM((RPW,), jnp.int32),
            pltpu.VMEM((2, CK, D), jnp.float32),
            pltpu.SemaphoreType.DMA((2,)),
        ],
    )
    def k(table_hbm, idx_hbm, out_hbm, idx_v, rows_v, sem):
        wid = lax.axis_index("s") * NC + lax.axis_index("c")
        base0 = wid * RPW
        pltpu.sync_copy(idx_hbm.at[pl.ds(base0, RPW)], idx_v)

        def start(c, slot):
            pltpu.async_copy(table_hbm.at[idx_v.at[pl.ds(c * CK, CK)]],
                             rows_v.at[slot], sem.at[slot])

        start(0, 0)

        def body(c, carry):
            slot = lax.rem(c, 2)

            @pl.when(c + 1 < nch)
            def _():
                start(c + 1, lax.rem(c + 1, 2))

            pltpu.make_async_copy(
                table_hbm.at[idx_v.at[pl.ds(c * CK, CK)]],
                rows_v.at[slot], sem.at[slot]).wait()
            pltpu.sync_copy(rows_v.at[slot],
                            out_hbm.at[pl.ds(base0 + c * CK, CK)])
            return carry

        lax.fori_loop(0, nch, body, 0)

    return k(table, idx)


def _silu(x):
    return x / (1.0 + jnp.exp(-x))


def _nbr_kernel(pos8, posT, batch_c, batch_cT):
    RB = 256
    W = 1024
    INF = float(jnp.inf)

    def body(pos_ref, posT_ref, bc_ref, bcT_ref, nbr_ref, msk_ref):
        b = pl.program_id(0)
        r0 = b * RB
        bcol = bc_ref[...]
        rsel = lax.broadcasted_iota(jnp.int32, (N, 1), 0) == r0
        b0 = jnp.sum(jnp.where(rsel, bcol, 0))
        w = jnp.sum((bcol < b0).astype(jnp.int32))
        w = jnp.minimum((w // 128) * 128, N - W)
        w = pl.multiple_of(w, 128)

        pr = pos_ref[pl.ds(r0, RB), :]
        pcT = posT_ref[:, pl.ds(w, W)]
        sq_r = pr[:, 0:1] ** 2 + pr[:, 1:2] ** 2 + pr[:, 2:3] ** 2
        sq_c = pcT[0:1, :] ** 2 + pcT[1:2, :] ** 2 + pcT[2:3, :] ** 2
        dot = jax.lax.dot_general(
            pr, pcT, (((1,), (0,)), ((), ())),
            precision=jax.lax.Precision.HIGHEST)
        d2 = (sq_r + sq_c) - 2.0 * dot
        d = jnp.sqrt(jnp.maximum(d2, 0.0))

        br = bc_ref[pl.ds(r0, RB), :]
        bcT_w = bcT_ref[:, pl.ds(w, W)]
        colidx = w + lax.broadcasted_iota(jnp.int32, (1, W), 1)
        rowidx = r0 + lax.broadcasted_iota(jnp.int32, (RB, 1), 0)
        valid = (br == bcT_w) & (d < CUT) & (colidx != rowidx)
        dw = jnp.where(valid, d, INF)

        lane = lax.broadcasted_iota(jnp.int32, (1, SLOTS), 1)
        BIG = jnp.int32(2 ** 30)

        def step(t, carry):
            dwc, nbr, msk = carry
            m = jnp.min(dwc, axis=1, keepdims=True)
            am = jnp.min(jnp.where(dwc == m, colidx, BIG),
                         axis=1, keepdims=True)
            has = m < INF
            amc = jnp.where(has, am, 0)
            onec = lane == t
            nbr = jnp.where(onec, amc, nbr)
            msk = jnp.where(onec, has.astype(jnp.float32), msk)
            dwc = jnp.where(colidx == am, INF, dwc)
            return dwc, nbr, msk

        nbr0 = jnp.zeros((RB, SLOTS), jnp.int32)
        msk0 = jnp.zeros((RB, SLOTS), jnp.float32)
        _, nbr, msk = lax.fori_loop(0, SLOTS, step, (dw, nbr0, msk0))
        nbr_ref[...] = nbr
        msk_ref[...] = msk

    full = lambda shape: pl.BlockSpec(shape, lambda b: tuple(0 for _ in shape))
    return pl.pallas_call(
        body,
        grid=(N // RB,),
        in_specs=[full((N, 8)), full((8, N)), full((N, 1)), full((1, N))],
        out_specs=[pl.BlockSpec((RB, SLOTS), lambda b: (b, 0)),
                   pl.BlockSpec((RB, SLOTS), lambda b: (b, 0))],
        out_shape=[jax.ShapeDtypeStruct((N, SLOTS), jnp.int32),
                   jax.ShapeDtypeStruct((N, SLOTS), jnp.float32)],
        interpret=_INTERP,
    )(pos8, posT, batch_c, batch_cT)


def _edge_pre_kernel(ws, pos8, posw, src_e, means, betas):
    NB = 32
    EB = NB * SLOTS

    def body(ws_ref, pos_ref, w0, w1, w2, w3, w4, src_ref,
             mean_ref, beta_ref, f_ref, d4_ref, c_ref):
        p = pos_ref[...]
        base = ws_ref[pl.program_id(0)] * 128
        srcv = src_ref[...]
        lane = lax.broadcasted_iota(jnp.int32, (1, 128), 1)
        psf = jnp.zeros((EB, 128), jnp.float32)
        for w_ref in (w0, w1, w2, w3, w4):
            oh = (srcv == base + lane).astype(jnp.float32)
            psf = psf + jax.lax.dot_general(
                oh, w_ref[...], (((1,), (0,)), ((), ())),
                precision=jax.lax.Precision.HIGHEST)
            base = base + 128
        prep = jnp.broadcast_to(p[:, None, :], (NB, SLOTS, 8))
        prep = prep.reshape(EB, 8)
        evx = psf[:, 0:1] - prep[:, 0:1]
        evy = psf[:, 1:2] - prep[:, 1:2]
        evz = psf[:, 2:3] - prep[:, 2:3]
        wgt = jnp.sqrt(evx * evx + evy * evy + evz * evz)
        wsafe = jnp.where(wgt > 0.0, wgt, 1.0)
        zero = jnp.zeros((EB, 1), jnp.float32)
        d4_ref[...] = jnp.concatenate(
            [evx / wsafe, evy / wsafe, evz / wsafe, zero], axis=1)
        cosc = 0.5 * (jnp.cos(wgt * (jnp.pi / CUT)) + 1.0)
        cosc = cosc * (wgt < CUT).astype(jnp.float32)
        c_ref[...] = cosc
        mu = mean_ref[...]
        be = beta_ref[...]
        t = jnp.exp(-wgt) - mu
        f_ref[...] = cosc * jnp.exp(-be * t * t)

    full = lambda shape: pl.BlockSpec(shape, lambda b, w: tuple(0 for _ in shape))
    win = lambda k: pl.BlockSpec((128, 128), lambda b, w: (w[b] + k, 0))
    grid_spec = pltpu.PrefetchScalarGridSpec(
        num_scalar_prefetch=1,
        grid=(N // NB,),
        in_specs=[pl.BlockSpec((NB, 8), lambda b, w: (b, 0)),
                  win(0), win(1), win(2), win(3), win(4),
                  pl.BlockSpec((EB, 1), lambda b, w: (b, 0)),
                  full((1, RBF)), full((1, RBF))],
        out_specs=[pl.BlockSpec((EB, RBF), lambda b, w: (b, 0)),
                   pl.BlockSpec((EB, 4), lambda b, w: (b, 0)),
                   pl.BlockSpec((EB, 1), lambda b, w: (b, 0))],
    )
    return pl.pallas_call(
        body,
        grid_spec=grid_spec,
        out_shape=[jax.ShapeDtypeStruct((E, RBF), jnp.float32),
                   jax.ShapeDtypeStruct((E, 4), jnp.float32),
                   jax.ShapeDtypeStruct((E, 1), jnp.float32)],
        interpret=_INTERP,
    )(ws, pos8, posw, posw, posw, posw, posw, src_e, means, betas)


def _nbr_embed_kernel(ws, f, xz, src_e, cvec, mvec, x0, dw, db, cw, cb):
    NB = 32
    EB = NB * SLOTS

    def body(ws_ref, f_ref, w0, w1, w2, w3, w4, src_ref, c_ref, m_ref,
             x0_ref, dw_ref, db_ref, cw_ref, cb_ref, out_ref):
        base = ws_ref[pl.program_id(0)] * 128
        srcv = src_ref[...]
        lane = lax.broadcasted_iota(jnp.int32, (1, 128), 1)
        xz_s = jnp.zeros((EB, H), jnp.float32)
        for w_ref in (w0, w1, w2, w3, w4):
            oh = (srcv == base + lane).astype(jnp.float32)
            xz_s = xz_s + jax.lax.dot_general(
                oh, w_ref[...], (((1,), (0,)), ((), ())),
                precision=jax.lax.Precision.HIGHEST)
            base = base + 128
        Wm = jnp.dot(f_ref[...], dw_ref[...]) + db_ref[...]
        Wm = Wm * c_ref[...] * m_ref[...]
        msgs = Wm * xz_s
        agg = msgs.reshape(NB, SLOTS, H).sum(axis=1)
        xcat = jnp.concatenate([x0_ref[...], agg], axis=1)
        out_ref[...] = jnp.dot(xcat, cw_ref[...]) + cb_ref[...]

    full = lambda shape: pl.BlockSpec(shape, lambda b, w: tuple(0 for _ in shape))
    win = lambda k: pl.BlockSpec((128, H), lambda b, w: (w[b] + k, 0))
    grid_spec = pltpu.PrefetchScalarGridSpec(
        num_scalar_prefetch=1,
        grid=(N // NB,),
        in_specs=[pl.BlockSpec((EB, RBF), lambda b, w: (b, 0)),
                  win(0), win(1), win(2), win(3), win(4),
                  pl.BlockSpec((EB, 1), lambda b, w: (b, 0)),
                  pl.BlockSpec((EB, 1), lambda b, w: (b, 0)),
                  pl.BlockSpec((EB, 1), lambda b, w: (b, 0)),
                  pl.BlockSpec((NB, H), lambda b, w: (b, 0)),
                  full((RBF, H)), full((1, H)),
                  full((2 * H, H)), full((1, H))],
        out_specs=pl.BlockSpec((NB, H), lambda b, w: (b, 0)),
    )
    return pl.pallas_call(
        body,
        grid_spec=grid_spec,
        out_shape=jax.ShapeDtypeStruct((N, H), jnp.float32),
        interpret=_INTERP,
    )(ws, f, xz, xz, xz, xz, xz, src_e, cvec, mvec, x0, dw, db, cw, cb)


def _layer_norm(x, w, b):
    mu = jnp.mean(x, axis=-1, keepdims=True)
    var = jnp.mean((x - mu) ** 2, axis=-1, keepdims=True)
    return (x - mu) / jnp.sqrt(var + 1e-5) * w + b


def _layer_pre_kernel(x, vecS, lnw, lnb, qw, qb, kw, kb, vw, vb, vecw):
    NB = 256

    def body(x_ref, vec_ref, lnw_ref, lnb_ref, qw_ref, qb_ref, kw_ref,
             kb_ref, vw_ref, vb_ref, vecw_ref,
             q_ref, tab_ref, vdot_ref, vec3_ref):
        xn = _layer_norm(x_ref[...], lnw_ref[...], lnb_ref[...])
        q = jnp.dot(xn, qw_ref[...]) + qb_ref[...]
        k = jnp.dot(xn, kw_ref[...]) + kb_ref[...]
        v = jnp.dot(xn, vw_ref[...]) + vb_ref[...]
        q_ref[...] = q
        vec = vec_ref[...]
        vdot = jnp.zeros((NB, H), jnp.float32)
        vecw_v = vecw_ref[...]
        for a in range(3):
            vp = jnp.dot(vec[a], vecw_v)
            vdot = vdot + vp[:, :H] * vp[:, H:2 * H]
            vec3_ref[a, :, :] = vp[:, 2 * H:]
        vdot_ref[...] = vdot
        tab_ref[...] = jnp.concatenate(
            [k, v, vec[0], vec[1], vec[2]], axis=1).astype(jnp.bfloat16)

    full = lambda shape: pl.BlockSpec(shape, lambda b: tuple(0 for _ in shape))
    return pl.pallas_call(
        body,
        grid=(N // NB,),
        in_specs=[pl.BlockSpec((NB, H), lambda b: (b, 0)),
                  pl.BlockSpec((3, NB, H), lambda b: (0, b, 0)),
                  full((1, H)), full((1, H)),
                  full((H, H)), full((1, H)),
                  full((H, H)), full((1, H)),
                  full((H, 3 * H)), full((1, 3 * H)),
                  full((H, 3 * H))],
        out_specs=[pl.BlockSpec((NB, H), lambda b: (b, 0)),
                   pl.BlockSpec((NB, 7 * H), lambda b: (b, 0)),
                   pl.BlockSpec((NB, H), lambda b: (b, 0)),
                   pl.BlockSpec((3, NB, H), lambda b: (0, b, 0))],
        out_shape=[jax.ShapeDtypeStruct((N, H), jnp.float32),
                   jax.ShapeDtypeStruct((N, 7 * H), jnp.bfloat16),
                   jax.ShapeDtypeStruct((N, H), jnp.float32),
                   jax.ShapeDtypeStruct((3, N, H), jnp.float32)],
        interpret=_INTERP,
    )(x, vecS, lnw, lnb, qw, qb, kw, kb, vw, vb, vecw)


def _win_starts_kernel(batch32, batch_cT):
    NBLK = N // 32
    WROWS = 640

    def body(b32_ref, bcT_ref, ws_ref):
        firsts = b32_ref[:, 0:1]
        cmp = (bcT_ref[...] < firsts).astype(jnp.int32)
        cnt = cmp.sum(axis=1, keepdims=True)
        ws = jnp.minimum((cnt // 128) * 128, N - WROWS)
        ws_ref[...] = ws // 128

    full = lambda shape: pl.BlockSpec(shape, lambda: tuple(0 for _ in shape))
    return pl.pallas_call(
        body,
        in_specs=[full((NBLK, 32)), full((1, N))],
        out_specs=full((NBLK, 1)),
        out_shape=jax.ShapeDtypeStruct((NBLK, 1), jnp.int32),
        interpret=_INTERP,
    )(batch32, batch_cT)


def _layer_edge_kernel(ws, q, tab, src_e, f, cvec, mvec, d4, vdot, vec3S,
                       x, vecS, dkw, dkb, dvw, dvb, ow, ob):
    NB = 32
    EB = NB * SLOTS

    def body(ws_ref, q_ref, t0_ref, t1_ref, t2_ref, t3_ref, t4_ref,
             src_ref, f_ref, c_ref, m_ref, d4_ref, vdot_ref,
             vec3_ref, x_ref, vec_ref, dkw_ref, dkb_ref, dvw_ref,
             dvb_ref, ow_ref, ob_ref, xo_ref, veco_ref):
        fv = f_ref[...].astype(jnp.bfloat16)
        dk = _silu(jax.lax.dot_general(
            fv, dkw_ref[...].astype(jnp.bfloat16), (((1,), (0,)), ((), ())),
            preferred_element_type=jnp.float32) + dkb_ref[...])
        dv = _silu(jax.lax.dot_general(
            fv, dvw_ref[...].astype(jnp.bfloat16), (((1,), (0,)), ((), ())),
            preferred_element_type=jnp.float32) + dvb_ref[...])
        base = ws_ref[pl.program_id(0)] * 128
        srcv = src_ref[...]
        lane = lax.broadcasted_iota(jnp.int32, (1, 128), 1)
        g = jnp.zeros((EB, 7 * H), jnp.float32)
        for t_ref in (t0_ref, t1_ref, t2_ref, t3_ref, t4_ref):
            oh = (srcv == base + lane).astype(jnp.bfloat16)
            g = g + jax.lax.dot_general(
                oh, t_ref[...], (((1,), (0,)), ((), ())),
                preferred_element_type=jnp.float32)
            base = base + 128
        qv = q_ref[...]
        qrep = jnp.broadcast_to(qv[:, None, :], (NB, SLOTS, H)).reshape(EB, H)
        prod = g[:, :H] * dk * qrep
        hsel = (lax.broadcasted_iota(jnp.int32, (H, NHEAD), 0) // HDIM ==
                lax.broadcasted_iota(jnp.int32, (H, NHEAD), 1)
                ).astype(jnp.float32)
        attn = jnp.dot(prod, hsel)
        attn = _silu(attn) * c_ref[...] * m_ref[...]
        attn_exp = jnp.dot(attn, hsel.T)
        vj = g[:, H:4 * H] * dv
        xmsg = vj[:, :H] * attn_exp
        x_agg = xmsg.reshape(NB, SLOTS, H).sum(axis=1)
        o = jnp.dot(x_agg, ow_ref[...]) + ob_ref[...]
        o1, o2, o3 = o[:, :H], o[:, H:2 * H], o[:, 2 * H:]
        xo_ref[...] = x_ref[...] + vdot_ref[...] * o2 + o3
        v1m = vj[:, H:2 * H]
        v2m = vj[:, 2 * H:]
        mv = m_ref[...]
        d4v = d4_ref[...]
        for a in range(3):
            vmsg = (g[:, (4 + a) * H:(5 + a) * H] * v1m
                    + v2m * d4v[:, a:a + 1]) * mv
            vagg = vmsg.reshape(NB, SLOTS, H).sum(axis=1)
            veco_ref[a, :, :] = (vec_ref[a] + vec3_ref[a] * o1 + vagg)

    full = lambda shape: pl.BlockSpec(shape, lambda b, w: tuple(0 for _ in shape))
    win = lambda k: pl.BlockSpec((128, 7 * H), lambda b, w: (w[b] + k, 0))
    grid_spec = pltpu.PrefetchScalarGridSpec(
        num_scalar_prefetch=1,
        grid=(N // NB,),
        in_specs=[pl.BlockSpec((NB, H), lambda b, w: (b, 0)),
                  win(0), win(1), win(2), win(3), win(4),
                  pl.BlockSpec((EB, 1), lambda b, w: (b, 0)),
                  pl.BlockSpec((EB, RBF), lambda b, w: (b, 0)),
                  pl.BlockSpec((EB, 1), lambda b, w: (b, 0)),
                  pl.BlockSpec((EB, 1), lambda b, w: (b, 0)),
                  pl.BlockSpec((EB, 4), lambda b, w: (b, 0)),
                  pl.BlockSpec((NB, H), lambda b, w: (b, 0)),
                  pl.BlockSpec((3, NB, H), lambda b, w: (0, b, 0)),
                  pl.BlockSpec((NB, H), lambda b, w: (b, 0)),
                  pl.BlockSpec((3, NB, H), lambda b, w: (0, b, 0)),
                  full((RBF, H)), full((1, H)),
                  full((RBF, 3 * H)), full((1, 3 * H)),
                  full((H, 3 * H)), full((1, 3 * H))],
        out_specs=[pl.BlockSpec((NB, H), lambda b, w: (b, 0)),
                   pl.BlockSpec((3, NB, H), lambda b, w: (0, b, 0))],
    )
    return pl.pallas_call(
        body,
        grid_spec=grid_spec,
        out_shape=[jax.ShapeDtypeStruct((N, H), jnp.float32),
                   jax.ShapeDtypeStruct((3, N, H), jnp.float32)],
        interpret=_INTERP,
    )(ws, q, tab, tab, tab, tab, tab, src_e, f, cvec, mvec, d4,
      vdot, vec3S, x, vecS, dkw, dkb, dvw, dvb, ow, ob)


def _final_ln_kernel(x, w, b):
    NB = 512

    def body(x_ref, w_ref, b_ref, o_ref):
        o_ref[...] = _layer_norm(x_ref[...], w_ref[...], b_ref[...])

    full = lambda shape: pl.BlockSpec(shape, lambda b: tuple(0 for _ in shape))
    return pl.pallas_call(
        body,
        grid=(N // NB,),
        in_specs=[pl.BlockSpec((NB, H), lambda b: (b, 0)),
                  full((1, H)), full((1, H))],
        out_specs=pl.BlockSpec((NB, H), lambda b: (b, 0)),
        out_shape=jax.ShapeDtypeStruct((N, H), jnp.float32),
        interpret=_INTERP,
    )(x, w, b)


def _vperm():
    j = jnp.arange(3 * H)
    p, jj = j // H, j % H
    return 96 * (jj // HDIM) + HDIM * p + (jj % HDIM)


def kernel(z, pos, batch, params):
    pos = pos.astype(jnp.float32)
    z = z.astype(jnp.int32)
    batch = batch.astype(jnp.int32)

    pos8 = jnp.pad(pos, ((0, 0), (0, 5)))
    posT = pos8.T
    batch_c = batch.reshape(N, 1)
    batch_cT = batch.reshape(1, N)

    nbr, mask = _nbr_kernel(pos8, posT, batch_c, batch_cT)
    src = nbr.reshape(E)
    src_e = nbr.reshape(E, 1)
    mask_e = mask.reshape(E, 1)
    ws = _win_starts_kernel(batch.reshape(N // 32, 32), batch_cT).reshape(N // 32)

    embcat = jnp.concatenate(
        [params['embedding'], params['nb_embedding']], axis=1)
    emb = _sc_gather(embcat, z)
    x0 = emb[:, :H]
    xz = emb[:, H:]

    means = params['means'].reshape(1, RBF)
    betas = params['betas'].reshape(1, RBF)
    posw = jnp.pad(pos, ((0, 0), (0, 125)))
    f, d4, cvec = _edge_pre_kernel(ws, pos8, posw, src_e, means, betas)

    x = _nbr_embed_kernel(
        ws, f, xz, src_e, cvec, mask_e, x0,
        params['nb_dist_w'], params['nb_dist_b'].reshape(1, H),
        params['nb_comb_w'], params['nb_comb_b'].reshape(1, H))

    vecS = jnp.zeros((3, N, H), jnp.float32)
    vp = _vperm()
    for lp in params['layers']:
        q, tab, vdot, vec3S = _layer_pre_kernel(
            x, vecS,
            lp['ln_w'].reshape(1, H), lp['ln_b'].reshape(1, H),
            lp['q_w'], lp['q_b'].reshape(1, H),
            lp['k_w'], lp['k_b'].reshape(1, H),
            lp['v_w'][:, vp], lp['v_b'][vp].reshape(1, 3 * H),
            lp['vec_w'])
        x, vecS = _layer_edge_kernel(
            ws, q, tab, src_e, f, cvec, mask_e, d4, vdot, vec3S, x, vecS,
            lp['dk_w'], lp['dk_b'].reshape(1, H),
            lp['dv_w'][:, vp], lp['dv_b'][vp].reshape(1, 3 * H),
            lp['o_w'], lp['o_b'].reshape(1, 3 * H))

    x = _final_ln_kernel(
        x, params['out_ln_w'].reshape(1, H), params['out_ln_b'].reshape(1, H))
    vec = jnp.transpose(vecS, (1, 0, 2))
    return x, vec
